# Initial kernel scaffold; baseline (speedup 1.0000x reference)
import jax
import jax.numpy as jnp
from jax.experimental import pallas as pl


def kernel(nodes, neigh_idx, features, W_init, b_init, W_final, b_final):
    raise NotImplementedError("write your pallas kernel here")


# speedup vs baseline: 1.1501x; 1.1501x over previous
import functools

import jax
import jax.numpy as jnp
from jax import lax
from jax.experimental import pallas as pl
from jax.experimental.pallas import tpu as pltpu
from jax.experimental.pallas import tpu_sc as plsc

N_NODES = 50000
D = 128
S = 10
BATCH = 50000

NC = 2
NS = 16
NW = NC * NS

CH = 128
CPW = 13
PER_W = CH * CPW
PAD = NW * PER_W

TB = 2048


def _s1_body(nidx_t_hbm, feat_hbm, nsum_hbm, idx_v, acc0_v, acc1_v,
             gsem0, gsem1, ssem0, ssem1):
    wid = lax.axis_index("s") * NC + lax.axis_index("c")
    base_c = wid * CPW

    pltpu.sync_copy(nidx_t_hbm.at[wid], idx_v)

    accs = (acc0_v, acc1_v)
    gsems = (gsem0, gsem1)
    ssems = (ssem0, ssem1)
    store = [None, None]
    for g in range(CPW):
        b = g % 2
        if store[b] is not None:
            store[b].wait()
        acc = accs[b]
        pltpu.async_copy(feat_hbm.at[idx_v.at[g]], acc,
                         gsems[b]).wait()
        drains = []
        for s in range(1, S):
            drains.append(pltpu.async_copy(
                feat_hbm.at[idx_v.at[s * CPW + g]], acc, gsems[b],
                add=True))
        for d in drains:
            d.wait()
        store[b] = pltpu.async_copy(
            acc, nsum_hbm.at[pl.ds((base_c + g) * CH, CH)], ssems[b])
    for d in store:
        if d is not None:
            d.wait()


@functools.partial(
    pl.kernel,
    out_type=jax.ShapeDtypeStruct((PAD, D), jnp.float32),
    mesh=plsc.VectorSubcoreMesh(core_axis_name="c", subcore_axis_name="s"),
    scratch_types=[
        pltpu.VMEM((S * CPW, CH), jnp.int32),
        pltpu.VMEM((CH, D), jnp.float32),
        pltpu.VMEM((CH, D), jnp.float32),
        pltpu.SemaphoreType.DMA,
        pltpu.SemaphoreType.DMA,
        pltpu.SemaphoreType.DMA,
        pltpu.SemaphoreType.DMA,
    ],
)
def _stage1(nidx_t_hbm, feat_hbm, nsum_hbm, idx_v, acc0_v, acc1_v,
            gsem0, gsem1, ssem0, ssem1):
    _s1_body(nidx_t_hbm, feat_hbm, nsum_hbm, idx_v, acc0_v, acc1_v,
             gsem0, gsem1, ssem0, ssem1)


def _s2_body(nodes_hbm, feat_hbm, nsum_hbm, self_hbm, nbr_hbm, nodes_v,
             sbuf0, sbuf1, nbuf0, nbuf1, sem0, sem1, osem0, osem1):
    wid = lax.axis_index("s") * NC + lax.axis_index("c")
    base_c = wid * CPW

    pltpu.sync_copy(nodes_hbm.at[wid], nodes_v)

    sbufs = (sbuf0, sbuf1)
    nbufs = (nbuf0, nbuf1)
    sems = (sem0, sem1)
    osems = (osem0, osem1)
    store = [None, None, None, None]
    for g in range(CPW):
        b = g % 2
        if store[2 * b] is not None:
            store[2 * b].wait()
            store[2 * b + 1].wait()
        idx = nodes_v.at[g]
        d1 = pltpu.async_copy(feat_hbm.at[idx], sbufs[b], sems[b])
        d2 = pltpu.async_copy(nsum_hbm.at[idx], nbufs[b], sems[b])
        d1.wait()
        d2.wait()
        rows = pl.ds((base_c + g) * CH, CH)
        store[2 * b] = pltpu.async_copy(sbufs[b], self_hbm.at[rows], osems[b])
        store[2 * b + 1] = pltpu.async_copy(nbufs[b], nbr_hbm.at[rows],
                                            osems[b])
    for d in store:
        if d is not None:
            d.wait()


@functools.partial(
    pl.kernel,
    out_type=(jax.ShapeDtypeStruct((PAD, D), jnp.float32),
              jax.ShapeDtypeStruct((PAD, D), jnp.float32)),
    mesh=plsc.VectorSubcoreMesh(core_axis_name="c", subcore_axis_name="s"),
    scratch_types=[
        pltpu.VMEM((CPW, CH), jnp.int32),
        pltpu.VMEM((CH, D), jnp.float32),
        pltpu.VMEM((CH, D), jnp.float32),
        pltpu.VMEM((CH, D), jnp.float32),
        pltpu.VMEM((CH, D), jnp.float32),
        pltpu.SemaphoreType.DMA,
        pltpu.SemaphoreType.DMA,
        pltpu.SemaphoreType.DMA,
        pltpu.SemaphoreType.DMA,
    ],
)
def _stage2(nodes_hbm, feat_hbm, nsum_hbm, self_hbm, nbr_hbm, nodes_v,
            sbuf0, sbuf1, nbuf0, nbuf1, sem0, sem1, osem0, osem1):
    _s2_body(nodes_hbm, feat_hbm, nsum_hbm, self_hbm, nbr_hbm, nodes_v,
             sbuf0, sbuf1, nbuf0, nbuf1, sem0, sem1, osem0, osem1)


def _prep_body(wi_ref, wf_ref, bi_ref, bf_ref, a_ref, c_ref, c0_ref):
    wi = wi_ref[...]
    wf1 = wf_ref[:D, :]
    wf2 = wf_ref[D:, :]
    a_ref[...] = jnp.dot(wi, wf1, preferred_element_type=jnp.float32)
    c_ref[...] = jnp.dot(wi, wf2, preferred_element_type=jnp.float32) * (
        1.0 / S)
    c0_ref[...] = (jnp.dot(bi_ref[...], wf1 + wf2,
                           preferred_element_type=jnp.float32) + bf_ref[...])


def _mm_body(self_ref, nbr_ref, a_ref, c_ref, c0_ref, o_ref):
    x = jnp.dot(self_ref[...], a_ref[...], preferred_element_type=jnp.float32)
    x = x + jnp.dot(nbr_ref[...], c_ref[...],
                    preferred_element_type=jnp.float32)
    x = x + c0_ref[...]
    o_ref[...] = x * (1.0 / (1.0 + jnp.exp(-x)))


def _fold_weights(w_init, w_final, b_init, b_final):
    return pl.pallas_call(
        _prep_body,
        out_shape=(jax.ShapeDtypeStruct((D, D), jnp.float32),
                   jax.ShapeDtypeStruct((D, D), jnp.float32),
                   jax.ShapeDtypeStruct((1, D), jnp.float32)),
    )(w_init, w_final, b_init.reshape(1, D), b_final.reshape(1, D))


def _matmul_swish(self_g, nbr_g, a, c, c0):
    grid = (PAD // TB,)
    return pl.pallas_call(
        _mm_body,
        grid=grid,
        in_specs=[
            pl.BlockSpec((TB, D), lambda i: (i, 0)),
            pl.BlockSpec((TB, D), lambda i: (i, 0)),
            pl.BlockSpec((D, D), lambda i: (0, 0)),
            pl.BlockSpec((D, D), lambda i: (0, 0)),
            pl.BlockSpec((1, D), lambda i: (0, 0)),
        ],
        out_specs=pl.BlockSpec((TB, D), lambda i: (i, 0)),
        out_shape=jax.ShapeDtypeStruct((PAD, D), jnp.float32),
    )(self_g, nbr_g, a, c, c0)


@jax.jit
def kernel(nodes, neigh_idx, features, W_init, b_init, W_final, b_final):
    nodes_p = jnp.pad(nodes.astype(jnp.int32), (0, PAD - BATCH))
    nidx_t = jnp.pad(neigh_idx.astype(jnp.int32),
                     ((0, PAD - N_NODES), (0, 0))).T
    nidx_w = nidx_t.reshape(S, NW, CPW * CH).transpose(1, 0, 2).reshape(
        NW, S * CPW, CH)
    nodes3 = nodes_p.reshape(NW, CPW, CH)

    nsum = _stage1(nidx_w, features)
    self_g, nbr_g = _stage2(nodes3, features, nsum)
    a, c, c0 = _fold_weights(W_init, W_final, b_init, b_final)
    out = _matmul_swish(self_g, nbr_g, a, c, c0)
    return out[:BATCH]


# speedup vs baseline: 1.1705x; 1.0178x over previous
import functools

import jax
import jax.numpy as jnp
from jax import lax
from jax.experimental import pallas as pl
from jax.experimental.pallas import tpu as pltpu
from jax.experimental.pallas import tpu_sc as plsc

N_NODES = 50000
D = 128
S = 10
BATCH = 50000

NC = 2
NS = 16
NW = NC * NS

CH = 128
CPW = 13
PER_W = CH * CPW
PAD = NW * PER_W

PLAN = ((0, 4, 0), (4, 2, 1), (6, 4, 0), (10, 2, 1), (12, 1, 1))

TB = 2048


def _worker_id():
    return lax.axis_index("s") * NC + lax.axis_index("c")


def _zero_rows(acc, nrows):
    zero = jnp.zeros((16,), jnp.float32)

    def body(r, _):
        for c in range(D // 16):
            acc[r, pl.ds(c * 16, 16)] = zero
        return 0

    lax.fori_loop(0, nrows, body, 0)


def _s1_body(nidx_hbm, feat_hbm, nsum_hbm, idx_v, acc0_v, acc1_v,
             gsem0, gsem1, ssem0, ssem1):
    wid = _worker_id()
    base_c = wid * CPW

    pltpu.sync_copy(nidx_hbm.at[wid], idx_v)

    accs = (acc0_v, acc1_v)
    gsems = (gsem0, gsem1)
    ssems = (ssem0, ssem1)
    store = [None, None]
    for g0, ck, b in PLAN:
        if store[b] is not None:
            store[b].wait()
        acc = accs[b]
        _zero_rows(acc, ck * CH)
        drains = []
        for c in range(ck):
            dst = acc.at[pl.ds(c * CH, CH)]
            for s in range(S):
                drains.append(pltpu.async_copy(
                    feat_hbm.at[idx_v.at[s * CPW + g0 + c]], dst, gsems[b],
                    add=True))
        for d in drains:
            d.wait()
        store[b] = pltpu.async_copy(
            acc.at[pl.ds(0, ck * CH)],
            nsum_hbm.at[pl.ds((base_c + g0) * CH, ck * CH)], ssems[b])
    for d in store:
        if d is not None:
            d.wait()


@functools.partial(
    pl.kernel,
    out_type=jax.ShapeDtypeStruct((PAD, D), jnp.float32),
    mesh=plsc.VectorSubcoreMesh(core_axis_name="c", subcore_axis_name="s"),
    scratch_types=[
        pltpu.VMEM((S * CPW, CH), jnp.int32),
        pltpu.VMEM((4 * CH, D), jnp.float32),
        pltpu.VMEM((2 * CH, D), jnp.float32),
        pltpu.SemaphoreType.DMA,
        pltpu.SemaphoreType.DMA,
        pltpu.SemaphoreType.DMA,
        pltpu.SemaphoreType.DMA,
    ],
)
def _stage1(nidx_hbm, feat_hbm, nsum_hbm, idx_v, acc0_v, acc1_v,
            gsem0, gsem1, ssem0, ssem1):
    _s1_body(nidx_hbm, feat_hbm, nsum_hbm, idx_v, acc0_v, acc1_v,
             gsem0, gsem1, ssem0, ssem1)


def _s2_body(nodes_hbm, pw_hbm, out_hbm, nodes_v, buf0, buf1,
             gsem0, gsem1, ssem0, ssem1):
    wid = _worker_id()
    base_c = wid * CPW

    pltpu.sync_copy(nodes_hbm.at[wid], nodes_v)

    bufs = (buf0, buf1)
    gsems = (gsem0, gsem1)
    ssems = (ssem0, ssem1)
    store = [None, None]
    for g0, ck, b in PLAN:
        if store[b] is not None:
            store[b].wait()
        buf = bufs[b]
        drains = []
        for c in range(ck):
            drains.append(pltpu.async_copy(
                pw_hbm.at[nodes_v.at[g0 + c]],
                buf.at[pl.ds(c * CH, CH)], gsems[b]))
        for d in drains:
            d.wait()
        store[b] = pltpu.async_copy(
            buf.at[pl.ds(0, ck * CH)],
            out_hbm.at[pl.ds((base_c + g0) * CH, ck * CH)], ssems[b])
    for d in store:
        if d is not None:
            d.wait()


@functools.partial(
    pl.kernel,
    out_type=jax.ShapeDtypeStruct((PAD, D), jnp.float32),
    mesh=plsc.VectorSubcoreMesh(core_axis_name="c", subcore_axis_name="s"),
    scratch_types=[
        pltpu.VMEM((CPW, CH), jnp.int32),
        pltpu.VMEM((4 * CH, D), jnp.float32),
        pltpu.VMEM((2 * CH, D), jnp.float32),
        pltpu.SemaphoreType.DMA,
        pltpu.SemaphoreType.DMA,
        pltpu.SemaphoreType.DMA,
        pltpu.SemaphoreType.DMA,
    ],
)
def _stage2(nodes_hbm, pw_hbm, out_hbm, nodes_v, buf0, buf1,
            gsem0, gsem1, ssem0, ssem1):
    _s2_body(nodes_hbm, pw_hbm, out_hbm, nodes_v, buf0, buf1,
             gsem0, gsem1, ssem0, ssem1)


def _prep_body(wi_ref, wf_ref, bi_ref, bf_ref, a_ref, c_ref, c0_ref):
    wi = wi_ref[...]
    wf1 = wf_ref[:D, :]
    wf2 = wf_ref[D:, :]
    a_ref[...] = jnp.dot(wi, wf1, preferred_element_type=jnp.float32)
    c_ref[...] = jnp.dot(wi, wf2, preferred_element_type=jnp.float32) * (
        1.0 / S)
    c0_ref[...] = (jnp.dot(bi_ref[...], wf1 + wf2,
                           preferred_element_type=jnp.float32) + bf_ref[...])


def _mm_body(feat_ref, nsum_ref, a_ref, c_ref, c0_ref, o_ref):
    x = jnp.dot(feat_ref[...], a_ref[...], preferred_element_type=jnp.float32)
    x = x + jnp.dot(nsum_ref[...], c_ref[...],
                    preferred_element_type=jnp.float32)
    x = x + c0_ref[...]
    o_ref[...] = x * (1.0 / (1.0 + jnp.exp(-x)))


def _fold_weights(w_init, w_final, b_init, b_final):
    return pl.pallas_call(
        _prep_body,
        out_shape=(jax.ShapeDtypeStruct((D, D), jnp.float32),
                   jax.ShapeDtypeStruct((D, D), jnp.float32),
                   jax.ShapeDtypeStruct((1, D), jnp.float32)),
    )(w_init, w_final, b_init.reshape(1, D), b_final.reshape(1, D))


def _matmul_swish(feats_p, nsum, a, c, c0):
    grid = (PAD // TB,)
    return pl.pallas_call(
        _mm_body,
        grid=grid,
        in_specs=[
            pl.BlockSpec((TB, D), lambda i: (i, 0)),
            pl.BlockSpec((TB, D), lambda i: (i, 0)),
            pl.BlockSpec((D, D), lambda i: (0, 0)),
            pl.BlockSpec((D, D), lambda i: (0, 0)),
            pl.BlockSpec((1, D), lambda i: (0, 0)),
        ],
        out_specs=pl.BlockSpec((TB, D), lambda i: (i, 0)),
        out_shape=jax.ShapeDtypeStruct((PAD, D), jnp.float32),
    )(feats_p, nsum, a, c, c0)


@jax.jit
def kernel(nodes, neigh_idx, features, W_init, b_init, W_final, b_final):
    nodes_p = jnp.pad(nodes.astype(jnp.int32), (0, PAD - BATCH))
    nidx_t = jnp.pad(neigh_idx.astype(jnp.int32),
                     ((0, PAD - N_NODES), (0, 0))).T
    nidx_w = nidx_t.reshape(S, NW, CPW * CH).transpose(1, 0, 2).reshape(
        NW, S * CPW, CH)
    nodes3 = nodes_p.reshape(NW, CPW, CH)
    feats_p = jnp.pad(features, ((0, PAD - N_NODES), (0, 0)))

    nsum = _stage1(nidx_w, features)
    a, c, c0 = _fold_weights(W_init, W_final, b_init, b_final)
    pw = _matmul_swish(feats_p, nsum, a, c, c0)
    out = _stage2(nodes3, pw)
    return out[:BATCH]
